# baseline (speedup 1.0000x reference)
import functools

import jax
import jax.numpy as jnp
from jax import lax
from jax.experimental import pallas as pl
from jax.experimental.pallas import tpu as pltpu
from jax.experimental.pallas import tpu_sc as plsc

N = 10000
E = 320000
D = 128
H = 8
DH = 16
HOP = 4
ALPHA = 0.15

NC = 2
NS = 16
NW = NC * NS
C = 80
NACC = 10240
ZC = 32

_f32 = jnp.float32
_i32 = jnp.int32


def _mesh():
    return plsc.VectorSubcoreMesh(core_axis_name="c", subcore_axis_name="s")


def _worker_id():
    return lax.axis_index("s") * NC + lax.axis_index("c")


def _lane(j):
    return jnp.zeros((16,), _i32) + j


@functools.lru_cache(maxsize=None)
def _sc_gather_kernel(n_rows, tab_lens):
    nt = len(tab_lens)
    per_w = n_rows // NW
    n = per_w // C

    def body(*refs):
        tabs = refs[:nt]
        idxs = refs[nt:2 * nt]
        outs = refs[2 * nt:3 * nt]
        idx_v = [[refs[3 * nt + 2 * t + b] for b in (0, 1)] for t in range(nt)]
        rows_v = [[refs[5 * nt + 2 * t + b] for b in (0, 1)] for t in range(nt)]
        lsem, gsem, osem = refs[7 * nt:7 * nt + 3]
        wid = _worker_id()

        def issue(i, b):
            base = wid * per_w + i * C
            for t in range(nt):
                pltpu.async_copy(idxs[t].at[pl.ds(base, C)], idx_v[t][b], lsem)

        def drain_lin(b):
            for t in range(nt):
                pltpu.make_async_copy(
                    idxs[t].at[pl.ds(0, C)], idx_v[t][b], lsem).wait()

        def gstart(b):
            for t in range(nt):
                pltpu.async_copy(tabs[t].at[idx_v[t][b]], rows_v[t][b], gsem)

        def gdrain(b):
            for t in range(nt):
                pltpu.make_async_copy(
                    tabs[t].at[pl.ds(0, C)], rows_v[t][b], gsem).wait()

        def ostart(i, b):
            base = wid * per_w + i * C
            for t in range(nt):
                pltpu.async_copy(rows_v[t][b], outs[t].at[pl.ds(base, C)], osem)

        def odrain(b):
            for t in range(nt):
                pltpu.make_async_copy(
                    rows_v[t][b], outs[t].at[pl.ds(0, C)], osem).wait()

        if n == 1:
            issue(0, 0)
            drain_lin(0)
            gstart(0)
            gdrain(0)
            ostart(0, 0)
            odrain(0)
            return

        issue(0, 0)
        drain_lin(0)
        gstart(0)
        issue(1, 1)

        def pair(g, carry):
            for b in (0, 1):
                i = 2 * g + b
                o = 1 - b

                @pl.when(i < n)
                def _():
                    gdrain(b)

                    @pl.when(i + 1 < n)
                    def _():
                        drain_lin(o)

                        @pl.when(i >= 1)
                        def _():
                            odrain(o)
                        gstart(o)
                    ostart(i, b)

                    @pl.when(i + 2 < n)
                    def _():
                        issue(i + 2, b)
            return carry

        lax.fori_loop(0, (n + 1) // 2, pair, 0)
        odrain((n - 2) % 2)
        odrain((n - 1) % 2)

    scratch = []
    for _ in range(nt):
        scratch += [pltpu.VMEM((C,), _i32)] * 2
    for _ in range(nt):
        scratch += [pltpu.VMEM((C, D), _f32)] * 2
    scratch += [pltpu.SemaphoreType.DMA] * 3
    return pl.kernel(
        body,
        out_type=[jax.ShapeDtypeStruct((n_rows, D), _f32)] * nt,
        mesh=_mesh(),
        scratch_types=scratch,
        name=f"sc_gather{nt}_{n_rows}",
    )


def _gather_rows(table, idx):
    return _sc_gather_kernel(idx.shape[0], (table.shape[0],))(table, idx)[0]


def _gather_rows_multi(tables, idxs):
    n_rows = idxs[0].shape[0]
    return _sc_gather_kernel(n_rows, tuple(t.shape[0] for t in tables))(
        *tables, *idxs)


@functools.lru_cache(maxsize=None)
def _sc_scatter_kernel(tab_len):
    per_w = E // NW
    n = per_w // C
    rpt = NACC // NS

    def body(tab, e16_hbm, src_hbm, dst_hbm, outm,
             sidx0, sidx1, didx0, didx1, e16_0, e16_1, rows_0, rows_1,
             stage, acc, lsem, gsem):
        sidx = [sidx0, sidx1]
        didx = [didx0, didx1]
        e16_v = [e16_0, e16_1]
        rows_v = [rows_0, rows_1]

        cid = lax.axis_index("c")
        sid = lax.axis_index("s")
        wid = sid * NC + cid
        row0 = sid * rpt

        def zrow(r, c):
            for j in range(D // 16):
                stage[r, pl.ds(16 * j, 16)] = jnp.zeros((16,), _f32)
            return c

        lax.fori_loop(0, ZC, zrow, 0)
        for z in range(rpt // ZC):
            pltpu.sync_copy(stage.at[:, :], acc.at[pl.ds(row0 + z * ZC, ZC)])
        plsc.subcore_barrier()

        def issue(i, b):
            base = wid * per_w + i * C
            pltpu.async_copy(src_hbm.at[pl.ds(base, C)], sidx[b], lsem)
            pltpu.async_copy(dst_hbm.at[pl.ds(base, C)], didx[b], lsem)
            pltpu.async_copy(e16_hbm.at[pl.ds(base, C)], e16_v[b], lsem)

        def drain_lin(b):
            pltpu.make_async_copy(src_hbm.at[pl.ds(0, C)], sidx[b], lsem).wait()
            pltpu.make_async_copy(dst_hbm.at[pl.ds(0, C)], didx[b], lsem).wait()
            pltpu.make_async_copy(e16_hbm.at[pl.ds(0, C)], e16_v[b], lsem).wait()

        def gstart(b):
            pltpu.async_copy(tab.at[sidx[b]], rows_v[b], gsem)

        def gdrain(b):
            pltpu.make_async_copy(tab.at[pl.ds(0, C)], rows_v[b], gsem).wait()

        issue(0, 0)
        drain_lin(0)
        gstart(0)
        issue(1, 1)

        def pair(g, carry):
            for b in (0, 1):
                i = 2 * g + b
                o = 1 - b

                @pl.when(i < n)
                def _():
                    gdrain(b)

                    @pl.when(i + 1 < n)
                    def _():
                        drain_lin(o)
                        gstart(o)

                    def mrow(r, cc):
                        erow = e16_v[b][r, :]
                        for j in range(H):
                            sl = pl.ds(16 * j, 16)
                            w = jnp.broadcast_to(erow[j], (16,))
                            rows_v[b][r, sl] = rows_v[b][r, sl] * w
                        return cc

                    lax.fori_loop(0, C, mrow, 0, unroll=2)
                    pltpu.sync_copy(rows_v[b], acc.at[didx[b]], add=True)

                    @pl.when(i + 2 < n)
                    def _():
                        issue(i + 2, b)
            return carry

        lax.fori_loop(0, (n + 1) // 2, pair, 0)
        plsc.subcore_barrier()

        for z in range(rpt // ZC):
            r0 = row0 + z * ZC
            pltpu.sync_copy(acc.at[pl.ds(r0, ZC)], stage)
            pltpu.sync_copy(stage, outm.at[pl.ds(cid * NACC + r0, ZC)])

    scratch = [pltpu.VMEM((C,), _i32)] * 4
    scratch += [pltpu.VMEM((C, 16), _f32)] * 2
    scratch += [pltpu.VMEM((C, D), _f32)] * 2
    scratch.append(pltpu.VMEM((ZC, D), _f32))
    scratch.append(pltpu.VMEM_SHARED((NACC, D), _f32))
    scratch += [pltpu.SemaphoreType.DMA] * 2
    return pl.kernel(
        body,
        out_type=jax.ShapeDtypeStruct((NC * NACC, D), _f32),
        mesh=_mesh(),
        scratch_types=scratch,
        name="sc_scatter",
    )


@functools.lru_cache(maxsize=None)
def _sc_scatter16_kernel():
    per_w = E // NW
    n_chunks = per_w // C
    rpt = NACC // NS
    zc16 = 128

    def body(e16_hbm, dst_hbm, out16, didx, e16_v, stage16, acc16, sem):
        cid = lax.axis_index("c")
        sid = lax.axis_index("s")
        wid = sid * NC + cid
        row0 = sid * rpt

        def zrow16(r, c):
            stage16[r, :] = jnp.zeros((16,), _f32)
            return c

        lax.fori_loop(0, zc16, zrow16, 0)
        for z in range(rpt // zc16):
            pltpu.sync_copy(stage16.at[:, :],
                            acc16.at[pl.ds(row0 + z * zc16, zc16)])
        plsc.subcore_barrier()

        def chunk(i, c):
            base = wid * per_w + i * C
            pltpu.sync_copy(dst_hbm.at[pl.ds(base, C)], didx)
            pltpu.sync_copy(e16_hbm.at[pl.ds(base, C)], e16_v)
            pltpu.sync_copy(e16_v, acc16.at[didx], add=True)
            return c

        lax.fori_loop(0, n_chunks, chunk, 0)
        plsc.subcore_barrier()
        for z in range(rpt // zc16):
            r0 = row0 + z * zc16
            pltpu.sync_copy(acc16.at[pl.ds(r0, zc16)], stage16)
            pltpu.sync_copy(stage16, out16.at[pl.ds(cid * NACC + r0, zc16)])

    return pl.kernel(
        body,
        out_type=jax.ShapeDtypeStruct((NC * NACC, 16), _f32),
        mesh=_mesh(),
        scratch_types=[
            pltpu.VMEM((C,), _i32),
            pltpu.VMEM((C, 16), _f32),
            pltpu.VMEM((zc16, 16), _f32),
            pltpu.VMEM_SHARED((NACC, 16), _f32),
            pltpu.SemaphoreType.DMA,
        ],
        name="sc_scatter_e16",
    )


def _scatter_vals(tab, e16, idx, dst):
    return _sc_scatter_kernel(tab.shape[0])(tab, e16, idx, dst)


def _scatter_d(e16, dst):
    return _sc_scatter16_kernel()(e16, dst)


def _sel(dtype=_f32):
    r = lax.broadcasted_iota(_i32, (D, H), 0) // DH
    c = lax.broadcasted_iota(_i32, (D, H), 1)
    return (r == c).astype(dtype)


def _selT():
    r = lax.broadcasted_iota(_i32, (H, D), 0)
    c = lax.broadcasted_iota(_i32, (H, D), 1) // DH
    return (r == c).astype(_f32)


def _sel16():
    r = lax.broadcasted_iota(_i32, (H, 16), 0)
    c = lax.broadcasted_iota(_i32, (H, 16), 1)
    return (r == c).astype(_f32)


def _tc_qkv1(x, wq, wk, wv, rel_table, wr):
    def body(x_r, wq_r, wk_r, wv_r, rt_r, wr_r, q_r, k_r, v_r, rp_r):
        x_ = x_r[...]
        q_r[...] = jnp.dot(x_, wq_r[...], preferred_element_type=_f32)
        k_r[...] = jnp.dot(x_, wk_r[...], preferred_element_type=_f32)
        v_r[...] = jnp.dot(x_, wv_r[...], preferred_element_type=_f32)
        rp_r[...] = jnp.dot(rt_r[...], wr_r[...], preferred_element_type=_f32)

    return pl.pallas_call(
        body,
        out_shape=[jax.ShapeDtypeStruct((N, D), _f32)] * 3
        + [jax.ShapeDtypeStruct(rel_table.shape, _f32)],
    )(x, wq, wk, wv, rel_table, wr)


def _tc_qkv2(x, wq, wk, wv):
    def body(x_r, wq_r, wk_r, wv_r, q_r, k_r, v_r):
        x_ = x_r[...]
        q_r[...] = jnp.dot(x_, wq_r[...], preferred_element_type=_f32)
        k_r[...] = jnp.dot(x_, wk_r[...], preferred_element_type=_f32)
        v_r[...] = jnp.dot(x_, wv_r[...], preferred_element_type=_f32)

    return pl.pallas_call(
        body, out_shape=[jax.ShapeDtypeStruct((N, D), _f32)] * 3
    )(x, wq, wk, wv)


_EB = 2560


def _tc_scores(q_rows, k_rows, rp_rows):
    has_rel = rp_rows is not None

    def body(*refs):
        if has_rel:
            q_r, k_r, rp_r, e16_r = refs
            ks = k_r[...] + rp_r[...]
        else:
            q_r, k_r, e16_r = refs
            ks = k_r[...]
        qk = q_r[...] * ks
        s = jnp.dot(qk, _sel(), preferred_element_type=_f32) * 0.25
        s = jnp.where(s >= 0, s, 0.2 * s)
        es = jnp.exp(s)
        e16_r[...] = jnp.dot(es, _sel16(), preferred_element_type=_f32)

    grid = E // _EB
    bspec = pl.BlockSpec((_EB, D), lambda i: (i, 0))
    in_specs = [bspec] * (3 if has_rel else 2)
    args = (q_rows, k_rows, rp_rows) if has_rel else (q_rows, k_rows)
    return pl.pallas_call(
        body,
        grid=(grid,),
        in_specs=in_specs,
        out_specs=pl.BlockSpec((_EB, 16), lambda i: (i, 0)),
        out_shape=jax.ShapeDtypeStruct((E, 16), _f32),
    )(*args)


def _tc_feat0(ev2, d2, ev2b=None):
    def body(*refs):
        if ev2b is None:
            ev_r, d_r, f_r, dinv_r = refs
        else:
            ev_r, evb_r, d_r, f_r, dinv_r = refs
        ev = ev_r[...]
        evs = ev[:N] + ev[NACC:NACC + N]
        if ev2b is not None:
            evb = evb_r[...]
            evs = evs + evb[:N] + evb[NACC:NACC + N]
        d16 = d_r[...][:N] + d_r[...][NACC:NACC + N]
        dinv = 1.0 / (d16[:, :H] + 1e-9)
        dinv_w = jnp.dot(dinv, _selT(), preferred_element_type=_f32)
        f_r[...] = evs * dinv_w
        dinv_r[...] = dinv_w

    args = (ev2, d2) if ev2b is None else (ev2, ev2b, d2)
    return pl.pallas_call(
        body, out_shape=[jax.ShapeDtypeStruct((N, D), _f32)] * 2
    )(*args)


def _tc_hop(m2, dinv_w, feat0):
    def body(m_r, di_r, f_r, h_r):
        m = m_r[...]
        ms = m[:N] + m[NACC:NACC + N]
        h_r[...] = (1.0 - ALPHA) * ms * di_r[...] + ALPHA * f_r[...]

    return pl.pallas_call(
        body, out_shape=jax.ShapeDtypeStruct((N, D), _f32)
    )(m2, dinv_w, feat0)


def _tc_post(h, x, wres, g, b):
    def body(h_r, x_r, w_r, g_r, b_r, o_r):
        o = h_r[...] + jnp.dot(x_r[...], w_r[...], preferred_element_type=_f32)
        mu = jnp.mean(o, axis=1, keepdims=True)
        cdev = o - mu
        var = jnp.mean(cdev * cdev, axis=1, keepdims=True)
        o = cdev * lax.rsqrt(var + 1e-5) * g_r[...] + b_r[...]
        o_r[...] = jnp.where(o > 0, o, jnp.exp(jnp.minimum(o, 0.0)) - 1.0)

    return pl.pallas_call(
        body, out_shape=jax.ShapeDtypeStruct((N, D), _f32)
    )(h, x, wres, g.reshape(1, D), b.reshape(1, D))


def _layer(x, src, dst, wq, wk, wv, wres, g, b, rel_ids=None,
           rel_table=None, wr=None):
    if rel_ids is not None:
        q, k, v, rp_tab = _tc_qkv1(x, wq, wk, wv, rel_table, wr)
        q_rows, k_rows, rp_rows = _gather_rows_multi(
            (q, k, rp_tab), (dst, src, rel_ids))
    else:
        q, k, v = _tc_qkv2(x, wq, wk, wv)
        rp_tab = rp_rows = None
        q_rows, k_rows = _gather_rows_multi((q, k), (dst, src))
    e16 = _tc_scores(q_rows, k_rows, rp_rows)
    ev2 = _scatter_vals(v, e16, src, dst)
    ev2b = _scatter_vals(rp_tab, e16, rel_ids, dst) if rp_tab is not None else None
    d2 = _scatter_d(e16, dst)
    feat0, dinv_w = _tc_feat0(ev2, d2, ev2b)
    h = feat0
    for _ in range(HOP):
        m2 = _scatter_vals(h, e16, src, dst)
        h = _tc_hop(m2, dinv_w, feat0)
    return _tc_post(h, x, wres, g, b)


def kernel(ent_ids, rel_ids, edge_index, batch_node_ids, params):
    p = params
    src = edge_index[0].astype(_i32)
    dst = edge_index[1].astype(_i32)
    rel_ids = rel_ids.astype(_i32)

    npad = NW * C * 4
    eidx = jnp.zeros((npad,), _i32).at[:N].set(ent_ids.astype(_i32))
    x = _gather_rows(p['ent_table'], eidx)[:N]

    x = _layer(x, src, dst, p['Wq1'], p['Wk1'], p['Wv1'], p['Wres1'],
               p['g1'], p['b1'], rel_ids=rel_ids,
               rel_table=p['rel_table'], wr=p['Wr1'])
    x = _layer(x, src, dst, p['Wq2'], p['Wk2'], p['Wv2'], p['Wres2'],
               p['g2'], p['b2'])

    bpad = NW * C
    bidx = jnp.zeros((bpad,), _i32).at[:batch_node_ids.shape[0]].set(
        batch_node_ids.astype(_i32))
    return _gather_rows(x, bidx)[:batch_node_ids.shape[0]]


# speedup vs baseline: 1.1029x; 1.1029x over previous
import functools

import jax
import jax.numpy as jnp
from jax import lax
from jax.experimental import pallas as pl
from jax.experimental.pallas import tpu as pltpu
from jax.experimental.pallas import tpu_sc as plsc

N = 10000
E = 320000
D = 128
H = 8
DH = 16
HOP = 4
ALPHA = 0.15

NC = 2
NS = 16
NW = NC * NS
C = 80
NACC = 10240
ZC = 32

_f32 = jnp.float32
_i32 = jnp.int32


def _mesh():
    return plsc.VectorSubcoreMesh(core_axis_name="c", subcore_axis_name="s")


def _worker_id():
    return lax.axis_index("s") * NC + lax.axis_index("c")


def _lane(j):
    return jnp.zeros((16,), _i32) + j


@functools.lru_cache(maxsize=None)
def _sc_gather_kernel(n_rows, tab_lens):
    nt = len(tab_lens)
    per_w = n_rows // NW
    n = per_w // C

    def body(*refs):
        tabs = refs[:nt]
        idxs = refs[nt:2 * nt]
        outs = refs[2 * nt:3 * nt]
        idx_v = [[refs[3 * nt + 2 * t + b] for b in (0, 1)] for t in range(nt)]
        rows_v = [[refs[5 * nt + 2 * t + b] for b in (0, 1)] for t in range(nt)]
        lsem, gsem, osem = refs[7 * nt:7 * nt + 3]
        wid = _worker_id()

        def issue(i, b):
            base = wid * per_w + i * C
            for t in range(nt):
                pltpu.async_copy(idxs[t].at[pl.ds(base, C)], idx_v[t][b], lsem)

        def drain_lin(b):
            for t in range(nt):
                pltpu.make_async_copy(
                    idxs[t].at[pl.ds(0, C)], idx_v[t][b], lsem).wait()

        def gstart(b):
            for t in range(nt):
                pltpu.async_copy(tabs[t].at[idx_v[t][b]], rows_v[t][b], gsem)

        def gdrain(b):
            for t in range(nt):
                pltpu.make_async_copy(
                    tabs[t].at[pl.ds(0, C)], rows_v[t][b], gsem).wait()

        def ostart(i, b):
            base = wid * per_w + i * C
            for t in range(nt):
                pltpu.async_copy(rows_v[t][b], outs[t].at[pl.ds(base, C)], osem)

        def odrain(b):
            for t in range(nt):
                pltpu.make_async_copy(
                    rows_v[t][b], outs[t].at[pl.ds(0, C)], osem).wait()

        if n == 1:
            issue(0, 0)
            drain_lin(0)
            gstart(0)
            gdrain(0)
            ostart(0, 0)
            odrain(0)
            return

        issue(0, 0)
        drain_lin(0)
        gstart(0)
        issue(1, 1)

        def pair(g, carry):
            for b in (0, 1):
                i = 2 * g + b
                o = 1 - b

                @pl.when(i < n)
                def _():
                    gdrain(b)

                    @pl.when(i + 1 < n)
                    def _():
                        drain_lin(o)

                        @pl.when(i >= 1)
                        def _():
                            odrain(o)
                        gstart(o)
                    ostart(i, b)

                    @pl.when(i + 2 < n)
                    def _():
                        issue(i + 2, b)
            return carry

        lax.fori_loop(0, (n + 1) // 2, pair, 0)
        odrain((n - 2) % 2)
        odrain((n - 1) % 2)

    scratch = []
    for _ in range(nt):
        scratch += [pltpu.VMEM((C,), _i32)] * 2
    for _ in range(nt):
        scratch += [pltpu.VMEM((C, D), _f32)] * 2
    scratch += [pltpu.SemaphoreType.DMA] * 3
    return pl.kernel(
        body,
        out_type=[jax.ShapeDtypeStruct((n_rows, D), _f32)] * nt,
        mesh=_mesh(),
        scratch_types=scratch,
        name=f"sc_gather{nt}_{n_rows}",
    )


def _gather_rows(table, idx):
    return _sc_gather_kernel(idx.shape[0], (table.shape[0],))(table, idx)[0]


def _gather_rows_multi(tables, idxs):
    n_rows = idxs[0].shape[0]
    return _sc_gather_kernel(n_rows, tuple(t.shape[0] for t in tables))(
        *tables, *idxs)


@functools.lru_cache(maxsize=None)
def _sc_scatter_kernel(tab_len):
    per_w = E // NW
    n = per_w // C
    rpt = NACC // NS

    def body(tab, e16_hbm, src_hbm, dst_hbm, outm,
             sidx0, sidx1, didx0, didx1, e16_0, e16_1, rows_0, rows_1,
             stage, acc, lsem, gsem):
        sidx = [sidx0, sidx1]
        didx = [didx0, didx1]
        e16_v = [e16_0, e16_1]
        rows_v = [rows_0, rows_1]

        cid = lax.axis_index("c")
        sid = lax.axis_index("s")
        wid = sid * NC + cid
        row0 = sid * rpt

        def zrow(r, c):
            for j in range(D // 16):
                stage[r, pl.ds(16 * j, 16)] = jnp.zeros((16,), _f32)
            return c

        lax.fori_loop(0, ZC, zrow, 0)
        for z in range(rpt // ZC):
            pltpu.sync_copy(stage.at[:, :], acc.at[pl.ds(row0 + z * ZC, ZC)])
        plsc.subcore_barrier()

        def issue(i, b):
            base = wid * per_w + i * C
            pltpu.async_copy(src_hbm.at[pl.ds(base, C)], sidx[b], lsem)
            pltpu.async_copy(dst_hbm.at[pl.ds(base, C)], didx[b], lsem)
            pltpu.async_copy(e16_hbm.at[pl.ds(base, C)], e16_v[b], lsem)

        def drain_lin(b):
            pltpu.make_async_copy(src_hbm.at[pl.ds(0, C)], sidx[b], lsem).wait()
            pltpu.make_async_copy(dst_hbm.at[pl.ds(0, C)], didx[b], lsem).wait()
            pltpu.make_async_copy(e16_hbm.at[pl.ds(0, C)], e16_v[b], lsem).wait()

        def gstart(b):
            pltpu.async_copy(tab.at[sidx[b]], rows_v[b], gsem)

        def gdrain(b):
            pltpu.make_async_copy(tab.at[pl.ds(0, C)], rows_v[b], gsem).wait()

        issue(0, 0)
        drain_lin(0)
        gstart(0)
        issue(1, 1)

        def pair(g, carry):
            for b in (0, 1):
                i = 2 * g + b
                o = 1 - b

                @pl.when(i < n)
                def _():
                    gdrain(b)

                    @pl.when(i + 1 < n)
                    def _():
                        drain_lin(o)
                        gstart(o)

                    def mrow(r, cc):
                        erow = e16_v[b][r, :]
                        for j in range(H):
                            sl = pl.ds(16 * j, 16)
                            w = jnp.broadcast_to(erow[j], (16,))
                            rows_v[b][r, sl] = rows_v[b][r, sl] * w
                        return cc

                    lax.fori_loop(0, C, mrow, 0)
                    pltpu.sync_copy(rows_v[b], acc.at[didx[b]], add=True)

                    @pl.when(i + 2 < n)
                    def _():
                        issue(i + 2, b)
            return carry

        lax.fori_loop(0, (n + 1) // 2, pair, 0)
        plsc.subcore_barrier()

        for z in range(rpt // ZC):
            r0 = row0 + z * ZC
            pltpu.sync_copy(acc.at[pl.ds(r0, ZC)], stage)
            pltpu.sync_copy(stage, outm.at[pl.ds(cid * NACC + r0, ZC)])

    scratch = [pltpu.VMEM((C,), _i32)] * 4
    scratch += [pltpu.VMEM((C, 16), _f32)] * 2
    scratch += [pltpu.VMEM((C, D), _f32)] * 2
    scratch.append(pltpu.VMEM((ZC, D), _f32))
    scratch.append(pltpu.VMEM_SHARED((NACC, D), _f32))
    scratch += [pltpu.SemaphoreType.DMA] * 2
    return pl.kernel(
        body,
        out_type=jax.ShapeDtypeStruct((NC * NACC, D), _f32),
        mesh=_mesh(),
        scratch_types=scratch,
        name="sc_scatter",
    )


@functools.lru_cache(maxsize=None)
def _sc_scatter16_kernel():
    per_w = E // NW
    n_chunks = per_w // C
    rpt = NACC // NS
    zc16 = 128

    def body(e16_hbm, dst_hbm, out16, didx, e16_v, stage16, acc16, sem):
        cid = lax.axis_index("c")
        sid = lax.axis_index("s")
        wid = sid * NC + cid
        row0 = sid * rpt

        def zrow16(r, c):
            stage16[r, :] = jnp.zeros((16,), _f32)
            return c

        lax.fori_loop(0, zc16, zrow16, 0)
        for z in range(rpt // zc16):
            pltpu.sync_copy(stage16.at[:, :],
                            acc16.at[pl.ds(row0 + z * zc16, zc16)])
        plsc.subcore_barrier()

        def chunk(i, c):
            base = wid * per_w + i * C
            pltpu.sync_copy(dst_hbm.at[pl.ds(base, C)], didx)
            pltpu.sync_copy(e16_hbm.at[pl.ds(base, C)], e16_v)
            pltpu.sync_copy(e16_v, acc16.at[didx], add=True)
            return c

        lax.fori_loop(0, n_chunks, chunk, 0)
        plsc.subcore_barrier()
        for z in range(rpt // zc16):
            r0 = row0 + z * zc16
            pltpu.sync_copy(acc16.at[pl.ds(r0, zc16)], stage16)
            pltpu.sync_copy(stage16, out16.at[pl.ds(cid * NACC + r0, zc16)])

    return pl.kernel(
        body,
        out_type=jax.ShapeDtypeStruct((NC * NACC, 16), _f32),
        mesh=_mesh(),
        scratch_types=[
            pltpu.VMEM((C,), _i32),
            pltpu.VMEM((C, 16), _f32),
            pltpu.VMEM((zc16, 16), _f32),
            pltpu.VMEM_SHARED((NACC, 16), _f32),
            pltpu.SemaphoreType.DMA,
        ],
        name="sc_scatter_e16",
    )


def _scatter_vals(tab, e16, idx, dst):
    return _sc_scatter_kernel(tab.shape[0])(tab, e16, idx, dst)


def _scatter_d(e16, dst):
    return _sc_scatter16_kernel()(e16, dst)


def _sel(dtype=_f32):
    r = lax.broadcasted_iota(_i32, (D, H), 0) // DH
    c = lax.broadcasted_iota(_i32, (D, H), 1)
    return (r == c).astype(dtype)


def _selT():
    r = lax.broadcasted_iota(_i32, (H, D), 0)
    c = lax.broadcasted_iota(_i32, (H, D), 1) // DH
    return (r == c).astype(_f32)


def _sel16():
    r = lax.broadcasted_iota(_i32, (H, 16), 0)
    c = lax.broadcasted_iota(_i32, (H, 16), 1)
    return (r == c).astype(_f32)


def _tc_qkv1(x, wq, wk, wv, rel_table, wr):
    def body(x_r, wq_r, wk_r, wv_r, rt_r, wr_r, q_r, k_r, v_r, rp_r):
        x_ = x_r[...]
        q_r[...] = jnp.dot(x_, wq_r[...], preferred_element_type=_f32)
        k_r[...] = jnp.dot(x_, wk_r[...], preferred_element_type=_f32)
        v_r[...] = jnp.dot(x_, wv_r[...], preferred_element_type=_f32)
        rp_r[...] = jnp.dot(rt_r[...], wr_r[...], preferred_element_type=_f32)

    return pl.pallas_call(
        body,
        out_shape=[jax.ShapeDtypeStruct((N, D), _f32)] * 3
        + [jax.ShapeDtypeStruct(rel_table.shape, _f32)],
    )(x, wq, wk, wv, rel_table, wr)


def _tc_qkv2(x, wq, wk, wv):
    def body(x_r, wq_r, wk_r, wv_r, q_r, k_r, v_r):
        x_ = x_r[...]
        q_r[...] = jnp.dot(x_, wq_r[...], preferred_element_type=_f32)
        k_r[...] = jnp.dot(x_, wk_r[...], preferred_element_type=_f32)
        v_r[...] = jnp.dot(x_, wv_r[...], preferred_element_type=_f32)

    return pl.pallas_call(
        body, out_shape=[jax.ShapeDtypeStruct((N, D), _f32)] * 3
    )(x, wq, wk, wv)


_EB = 2560


def _tc_scores(q_rows, k_rows, rp_rows):
    has_rel = rp_rows is not None

    def body(*refs):
        if has_rel:
            q_r, k_r, rp_r, e16_r = refs
            ks = k_r[...] + rp_r[...]
        else:
            q_r, k_r, e16_r = refs
            ks = k_r[...]
        qk = q_r[...] * ks
        s = jnp.dot(qk, _sel(), preferred_element_type=_f32) * 0.25
        s = jnp.where(s >= 0, s, 0.2 * s)
        es = jnp.exp(s)
        e16_r[...] = jnp.dot(es, _sel16(), preferred_element_type=_f32)

    grid = E // _EB
    bspec = pl.BlockSpec((_EB, D), lambda i: (i, 0))
    in_specs = [bspec] * (3 if has_rel else 2)
    args = (q_rows, k_rows, rp_rows) if has_rel else (q_rows, k_rows)
    return pl.pallas_call(
        body,
        grid=(grid,),
        in_specs=in_specs,
        out_specs=pl.BlockSpec((_EB, 16), lambda i: (i, 0)),
        out_shape=jax.ShapeDtypeStruct((E, 16), _f32),
    )(*args)


def _tc_feat0(ev2, d2, ev2b=None):
    def body(*refs):
        if ev2b is None:
            ev_r, d_r, f_r, dinv_r = refs
        else:
            ev_r, evb_r, d_r, f_r, dinv_r = refs
        ev = ev_r[...]
        evs = ev[:N] + ev[NACC:NACC + N]
        if ev2b is not None:
            evb = evb_r[...]
            evs = evs + evb[:N] + evb[NACC:NACC + N]
        d16 = d_r[...][:N] + d_r[...][NACC:NACC + N]
        dinv = 1.0 / (d16[:, :H] + 1e-9)
        dinv_w = jnp.dot(dinv, _selT(), preferred_element_type=_f32)
        f_r[...] = evs * dinv_w
        dinv_r[...] = dinv_w

    args = (ev2, d2) if ev2b is None else (ev2, ev2b, d2)
    return pl.pallas_call(
        body, out_shape=[jax.ShapeDtypeStruct((N, D), _f32)] * 2
    )(*args)


def _tc_hop(m2, dinv_w, feat0):
    def body(m_r, di_r, f_r, h_r):
        m = m_r[...]
        ms = m[:N] + m[NACC:NACC + N]
        h_r[...] = (1.0 - ALPHA) * ms * di_r[...] + ALPHA * f_r[...]

    return pl.pallas_call(
        body, out_shape=jax.ShapeDtypeStruct((N, D), _f32)
    )(m2, dinv_w, feat0)


def _tc_post(h, x, wres, g, b):
    def body(h_r, x_r, w_r, g_r, b_r, o_r):
        o = h_r[...] + jnp.dot(x_r[...], w_r[...], preferred_element_type=_f32)
        mu = jnp.mean(o, axis=1, keepdims=True)
        cdev = o - mu
        var = jnp.mean(cdev * cdev, axis=1, keepdims=True)
        o = cdev * lax.rsqrt(var + 1e-5) * g_r[...] + b_r[...]
        o_r[...] = jnp.where(o > 0, o, jnp.exp(jnp.minimum(o, 0.0)) - 1.0)

    return pl.pallas_call(
        body, out_shape=jax.ShapeDtypeStruct((N, D), _f32)
    )(h, x, wres, g.reshape(1, D), b.reshape(1, D))


def _layer(x, src, dst, wq, wk, wv, wres, g, b, rel_ids=None,
           rel_table=None, wr=None):
    if rel_ids is not None:
        q, k, v, rp_tab = _tc_qkv1(x, wq, wk, wv, rel_table, wr)
        q_rows, k_rows, rp_rows = _gather_rows_multi(
            (q, k, rp_tab), (dst, src, rel_ids))
    else:
        q, k, v = _tc_qkv2(x, wq, wk, wv)
        rp_tab = rp_rows = None
        q_rows, k_rows = _gather_rows_multi((q, k), (dst, src))
    e16 = _tc_scores(q_rows, k_rows, rp_rows)
    ev2 = _scatter_vals(v, e16, src, dst)
    ev2b = _scatter_vals(rp_tab, e16, rel_ids, dst) if rp_tab is not None else None
    d2 = _scatter_d(e16, dst)
    feat0, dinv_w = _tc_feat0(ev2, d2, ev2b)
    h = feat0
    for _ in range(HOP):
        m2 = _scatter_vals(h, e16, src, dst)
        h = _tc_hop(m2, dinv_w, feat0)
    return _tc_post(h, x, wres, g, b)


def kernel(ent_ids, rel_ids, edge_index, batch_node_ids, params):
    p = params
    src = edge_index[0].astype(_i32)
    dst = edge_index[1].astype(_i32)
    rel_ids = rel_ids.astype(_i32)

    npad = NW * C * 4
    eidx = jnp.zeros((npad,), _i32).at[:N].set(ent_ids.astype(_i32))
    x = _gather_rows(p['ent_table'], eidx)[:N]

    x = _layer(x, src, dst, p['Wq1'], p['Wk1'], p['Wv1'], p['Wres1'],
               p['g1'], p['b1'], rel_ids=rel_ids,
               rel_table=p['rel_table'], wr=p['Wr1'])
    x = _layer(x, src, dst, p['Wq2'], p['Wk2'], p['Wv2'], p['Wres2'],
               p['g2'], p['b2'])

    bpad = NW * C
    bidx = jnp.zeros((bpad,), _i32).at[:batch_node_ids.shape[0]].set(
        batch_node_ids.astype(_i32))
    return _gather_rows(x, bidx)[:batch_node_ids.shape[0]]


# speedup vs baseline: 1.1725x; 1.0631x over previous
import functools

import jax
import jax.numpy as jnp
from jax import lax
from jax.experimental import pallas as pl
from jax.experimental.pallas import tpu as pltpu
from jax.experimental.pallas import tpu_sc as plsc

N = 10000
E = 320000
D = 128
H = 8
DH = 16
HOP = 4
ALPHA = 0.15

NC = 2
NS = 16
NW = NC * NS
C = 80
NACC = 10240
ZC = 32

_f32 = jnp.float32
_i32 = jnp.int32


def _mesh():
    return plsc.VectorSubcoreMesh(core_axis_name="c", subcore_axis_name="s")


def _worker_id():
    return lax.axis_index("s") * NC + lax.axis_index("c")


def _lane(j):
    return jnp.zeros((16,), _i32) + j


@functools.lru_cache(maxsize=None)
def _sc_gather_kernel(n_rows, tab_lens):
    nt = len(tab_lens)
    per_w = n_rows // NW
    n = per_w // C

    def body(*refs):
        tabs = refs[:nt]
        idxs = refs[nt:2 * nt]
        outs = refs[2 * nt:3 * nt]
        idx_v = [[refs[3 * nt + 2 * t + b] for b in (0, 1)] for t in range(nt)]
        rows_v = [[refs[5 * nt + 2 * t + b] for b in (0, 1)] for t in range(nt)]
        lsem, gsem, osem = refs[7 * nt:7 * nt + 3]
        wid = _worker_id()

        def issue(i, b):
            base = wid * per_w + i * C
            for t in range(nt):
                pltpu.async_copy(idxs[t].at[pl.ds(base, C)], idx_v[t][b], lsem)

        def drain_lin(b):
            for t in range(nt):
                pltpu.make_async_copy(
                    idxs[t].at[pl.ds(0, C)], idx_v[t][b], lsem).wait()

        def gstart(b):
            for t in range(nt):
                pltpu.async_copy(tabs[t].at[idx_v[t][b]], rows_v[t][b], gsem)

        def gdrain(b):
            for t in range(nt):
                pltpu.make_async_copy(
                    tabs[t].at[pl.ds(0, C)], rows_v[t][b], gsem).wait()

        def ostart(i, b):
            base = wid * per_w + i * C
            for t in range(nt):
                pltpu.async_copy(rows_v[t][b], outs[t].at[pl.ds(base, C)], osem)

        def odrain(b):
            for t in range(nt):
                pltpu.make_async_copy(
                    rows_v[t][b], outs[t].at[pl.ds(0, C)], osem).wait()

        if n == 1:
            issue(0, 0)
            drain_lin(0)
            gstart(0)
            gdrain(0)
            ostart(0, 0)
            odrain(0)
            return

        issue(0, 0)
        drain_lin(0)
        gstart(0)
        issue(1, 1)

        def pair(g, carry):
            for b in (0, 1):
                i = 2 * g + b
                o = 1 - b

                @pl.when(i < n)
                def _():
                    gdrain(b)

                    @pl.when(i + 1 < n)
                    def _():
                        drain_lin(o)

                        @pl.when(i >= 1)
                        def _():
                            odrain(o)
                        gstart(o)
                    ostart(i, b)

                    @pl.when(i + 2 < n)
                    def _():
                        issue(i + 2, b)
            return carry

        lax.fori_loop(0, (n + 1) // 2, pair, 0)
        odrain((n - 2) % 2)
        odrain((n - 1) % 2)

    scratch = []
    for _ in range(nt):
        scratch += [pltpu.VMEM((C,), _i32)] * 2
    for _ in range(nt):
        scratch += [pltpu.VMEM((C, D), _f32)] * 2
    scratch += [pltpu.SemaphoreType.DMA] * 3
    return pl.kernel(
        body,
        out_type=[jax.ShapeDtypeStruct((n_rows, D), _f32)] * nt,
        mesh=_mesh(),
        scratch_types=scratch,
        name=f"sc_gather{nt}_{n_rows}",
    )


def _gather_rows(table, idx):
    return _sc_gather_kernel(idx.shape[0], (table.shape[0],))(table, idx)[0]


def _gather_rows_multi(tables, idxs):
    n_rows = idxs[0].shape[0]
    return _sc_gather_kernel(n_rows, tuple(t.shape[0] for t in tables))(
        *tables, *idxs)


@functools.lru_cache(maxsize=None)
def _sc_scatter_kernel(tab_len):
    per_w = E // NW
    n = per_w // C
    rpt = NACC // NS

    def body(tab, e16_hbm, src_hbm, dst_hbm, outm,
             sidx0, sidx1, didx0, didx1, e16_0, e16_1, rows_0, rows_1,
             stage, acc, lsem, gsem):
        sidx = [sidx0, sidx1]
        didx = [didx0, didx1]
        e16_v = [e16_0, e16_1]
        rows_v = [rows_0, rows_1]

        cid = lax.axis_index("c")
        sid = lax.axis_index("s")
        wid = sid * NC + cid
        row0 = sid * rpt

        def zrow(r, c):
            for j in range(D // 16):
                stage[r, pl.ds(16 * j, 16)] = jnp.zeros((16,), _f32)
            return c

        lax.fori_loop(0, ZC, zrow, 0)
        for z in range(rpt // ZC):
            pltpu.async_copy(stage.at[:, :], acc.at[pl.ds(row0 + z * ZC, ZC)],
                             lsem)
        for z in range(rpt // ZC):
            pltpu.make_async_copy(stage.at[:, :], acc.at[pl.ds(row0, ZC)],
                                  lsem).wait()
        plsc.subcore_barrier()

        def issue(i, b):
            base = wid * per_w + i * C
            pltpu.async_copy(src_hbm.at[pl.ds(base, C)], sidx[b], lsem)
            pltpu.async_copy(dst_hbm.at[pl.ds(base, C)], didx[b], lsem)
            pltpu.async_copy(e16_hbm.at[pl.ds(base, C)], e16_v[b], lsem)

        def drain_lin(b):
            pltpu.make_async_copy(src_hbm.at[pl.ds(0, C)], sidx[b], lsem).wait()
            pltpu.make_async_copy(dst_hbm.at[pl.ds(0, C)], didx[b], lsem).wait()
            pltpu.make_async_copy(e16_hbm.at[pl.ds(0, C)], e16_v[b], lsem).wait()

        def gstart(b):
            pltpu.async_copy(tab.at[sidx[b]], rows_v[b], gsem)

        def gdrain(b):
            pltpu.make_async_copy(tab.at[pl.ds(0, C)], rows_v[b], gsem).wait()

        issue(0, 0)
        drain_lin(0)
        gstart(0)
        issue(1, 1)

        def pair(g, carry):
            for b in (0, 1):
                i = 2 * g + b
                o = 1 - b

                @pl.when(i < n)
                def _():
                    gdrain(b)

                    @pl.when(i + 1 < n)
                    def _():
                        drain_lin(o)
                        gstart(o)

                    def mrow(r, cc):
                        erow = e16_v[b][r, :]
                        for j in range(H):
                            sl = pl.ds(16 * j, 16)
                            w = jnp.broadcast_to(erow[j], (16,))
                            rows_v[b][r, sl] = rows_v[b][r, sl] * w
                        return cc

                    lax.fori_loop(0, C, mrow, 0)
                    pltpu.sync_copy(rows_v[b], acc.at[didx[b]], add=True)

                    @pl.when(i + 2 < n)
                    def _():
                        issue(i + 2, b)
            return carry

        lax.fori_loop(0, (n + 1) // 2, pair, 0)
        plsc.subcore_barrier()

        pltpu.sync_copy(acc.at[pl.ds(row0, rpt)],
                        outm.at[pl.ds(cid * NACC + row0, rpt)])

    scratch = [pltpu.VMEM((C,), _i32)] * 4
    scratch += [pltpu.VMEM((C, 16), _f32)] * 2
    scratch += [pltpu.VMEM((C, D), _f32)] * 2
    scratch.append(pltpu.VMEM((ZC, D), _f32))
    scratch.append(pltpu.VMEM_SHARED((NACC, D), _f32))
    scratch += [pltpu.SemaphoreType.DMA] * 2
    return pl.kernel(
        body,
        out_type=jax.ShapeDtypeStruct((NC * NACC, D), _f32),
        mesh=_mesh(),
        scratch_types=scratch,
        name="sc_scatter",
    )


@functools.lru_cache(maxsize=None)
def _sc_scatter16_kernel():
    per_w = E // NW
    n = per_w // C
    rpt = NACC // NS
    zc16 = 128

    def body(e16_hbm, dst_hbm, out16, didx0, didx1, e16_0, e16_1,
             stage16, acc16, lsem):
        didx = [didx0, didx1]
        e16_v = [e16_0, e16_1]
        cid = lax.axis_index("c")
        sid = lax.axis_index("s")
        wid = sid * NC + cid
        row0 = sid * rpt

        def zrow16(r, c):
            stage16[r, :] = jnp.zeros((16,), _f32)
            return c

        lax.fori_loop(0, zc16, zrow16, 0)
        for z in range(rpt // zc16):
            pltpu.async_copy(stage16.at[:, :],
                             acc16.at[pl.ds(row0 + z * zc16, zc16)], lsem)
        for z in range(rpt // zc16):
            pltpu.make_async_copy(stage16.at[:, :],
                                  acc16.at[pl.ds(row0, zc16)], lsem).wait()
        plsc.subcore_barrier()

        def issue(i, b):
            base = wid * per_w + i * C
            pltpu.async_copy(dst_hbm.at[pl.ds(base, C)], didx[b], lsem)
            pltpu.async_copy(e16_hbm.at[pl.ds(base, C)], e16_v[b], lsem)

        def drain(b):
            pltpu.make_async_copy(dst_hbm.at[pl.ds(0, C)], didx[b], lsem).wait()
            pltpu.make_async_copy(e16_hbm.at[pl.ds(0, C)], e16_v[b], lsem).wait()

        issue(0, 0)
        issue(1, 1)

        def pair(g, carry):
            for b in (0, 1):
                i = 2 * g + b

                @pl.when(i < n)
                def _():
                    drain(b)
                    pltpu.sync_copy(e16_v[b], acc16.at[didx[b]], add=True)

                    @pl.when(i + 2 < n)
                    def _():
                        issue(i + 2, b)
            return carry

        lax.fori_loop(0, (n + 1) // 2, pair, 0)
        plsc.subcore_barrier()
        pltpu.sync_copy(acc16.at[pl.ds(row0, rpt)],
                        out16.at[pl.ds(cid * NACC + row0, rpt)])

    return pl.kernel(
        body,
        out_type=jax.ShapeDtypeStruct((NC * NACC, 16), _f32),
        mesh=_mesh(),
        scratch_types=[
            pltpu.VMEM((C,), _i32),
            pltpu.VMEM((C,), _i32),
            pltpu.VMEM((C, 16), _f32),
            pltpu.VMEM((C, 16), _f32),
            pltpu.VMEM((zc16, 16), _f32),
            pltpu.VMEM_SHARED((NACC, 16), _f32),
            pltpu.SemaphoreType.DMA,
        ],
        name="sc_scatter_e16",
    )


def _scatter_vals(tab, e16, idx, dst):
    return _sc_scatter_kernel(tab.shape[0])(tab, e16, idx, dst)


def _scatter_d(e16, dst):
    return _sc_scatter16_kernel()(e16, dst)


def _sel(dtype=_f32):
    r = lax.broadcasted_iota(_i32, (D, H), 0) // DH
    c = lax.broadcasted_iota(_i32, (D, H), 1)
    return (r == c).astype(dtype)


def _selT():
    r = lax.broadcasted_iota(_i32, (H, D), 0)
    c = lax.broadcasted_iota(_i32, (H, D), 1) // DH
    return (r == c).astype(_f32)


def _sel16():
    r = lax.broadcasted_iota(_i32, (H, 16), 0)
    c = lax.broadcasted_iota(_i32, (H, 16), 1)
    return (r == c).astype(_f32)


def _tc_qkv1(x, wq, wk, wv, rel_table, wr):
    def body(x_r, wq_r, wk_r, wv_r, rt_r, wr_r, q_r, k_r, v_r, rp_r):
        x_ = x_r[...]
        q_r[...] = jnp.dot(x_, wq_r[...], preferred_element_type=_f32)
        k_r[...] = jnp.dot(x_, wk_r[...], preferred_element_type=_f32)
        v_r[...] = jnp.dot(x_, wv_r[...], preferred_element_type=_f32)
        rp_r[...] = jnp.dot(rt_r[...], wr_r[...], preferred_element_type=_f32)

    return pl.pallas_call(
        body,
        out_shape=[jax.ShapeDtypeStruct((N, D), _f32)] * 3
        + [jax.ShapeDtypeStruct(rel_table.shape, _f32)],
    )(x, wq, wk, wv, rel_table, wr)


def _tc_qkv2(x, wq, wk, wv):
    def body(x_r, wq_r, wk_r, wv_r, q_r, k_r, v_r):
        x_ = x_r[...]
        q_r[...] = jnp.dot(x_, wq_r[...], preferred_element_type=_f32)
        k_r[...] = jnp.dot(x_, wk_r[...], preferred_element_type=_f32)
        v_r[...] = jnp.dot(x_, wv_r[...], preferred_element_type=_f32)

    return pl.pallas_call(
        body, out_shape=[jax.ShapeDtypeStruct((N, D), _f32)] * 3
    )(x, wq, wk, wv)


_EB = 2560


def _tc_scores(q_rows, k_rows, rp_rows):
    has_rel = rp_rows is not None

    def body(*refs):
        if has_rel:
            q_r, k_r, rp_r, e16_r = refs
            ks = k_r[...] + rp_r[...]
        else:
            q_r, k_r, e16_r = refs
            ks = k_r[...]
        qk = q_r[...] * ks
        s = jnp.dot(qk, _sel(), preferred_element_type=_f32) * 0.25
        s = jnp.where(s >= 0, s, 0.2 * s)
        es = jnp.exp(s)
        e16_r[...] = jnp.dot(es, _sel16(), preferred_element_type=_f32)

    grid = E // _EB
    bspec = pl.BlockSpec((_EB, D), lambda i: (i, 0))
    in_specs = [bspec] * (3 if has_rel else 2)
    args = (q_rows, k_rows, rp_rows) if has_rel else (q_rows, k_rows)
    return pl.pallas_call(
        body,
        grid=(grid,),
        in_specs=in_specs,
        out_specs=pl.BlockSpec((_EB, 16), lambda i: (i, 0)),
        out_shape=jax.ShapeDtypeStruct((E, 16), _f32),
    )(*args)


def _tc_feat0(ev2, d2, ev2b=None):
    def body(*refs):
        if ev2b is None:
            ev_r, d_r, f_r, dinv_r = refs
        else:
            ev_r, evb_r, d_r, f_r, dinv_r = refs
        ev = ev_r[...]
        evs = ev[:N] + ev[NACC:NACC + N]
        if ev2b is not None:
            evb = evb_r[...]
            evs = evs + evb[:N] + evb[NACC:NACC + N]
        d16 = d_r[...][:N] + d_r[...][NACC:NACC + N]
        dinv = 1.0 / (d16[:, :H] + 1e-9)
        dinv_w = jnp.dot(dinv, _selT(), preferred_element_type=_f32)
        f_r[...] = evs * dinv_w
        dinv_r[...] = dinv_w

    args = (ev2, d2) if ev2b is None else (ev2, ev2b, d2)
    return pl.pallas_call(
        body, out_shape=[jax.ShapeDtypeStruct((N, D), _f32)] * 2
    )(*args)


def _tc_hop(m2, dinv_w, feat0):
    def body(m_r, di_r, f_r, h_r):
        m = m_r[...]
        ms = m[:N] + m[NACC:NACC + N]
        h_r[...] = (1.0 - ALPHA) * ms * di_r[...] + ALPHA * f_r[...]

    return pl.pallas_call(
        body, out_shape=jax.ShapeDtypeStruct((N, D), _f32)
    )(m2, dinv_w, feat0)


def _tc_post(h, x, wres, g, b):
    def body(h_r, x_r, w_r, g_r, b_r, o_r):
        o = h_r[...] + jnp.dot(x_r[...], w_r[...], preferred_element_type=_f32)
        mu = jnp.mean(o, axis=1, keepdims=True)
        cdev = o - mu
        var = jnp.mean(cdev * cdev, axis=1, keepdims=True)
        o = cdev * lax.rsqrt(var + 1e-5) * g_r[...] + b_r[...]
        o_r[...] = jnp.where(o > 0, o, jnp.exp(jnp.minimum(o, 0.0)) - 1.0)

    return pl.pallas_call(
        body, out_shape=jax.ShapeDtypeStruct((N, D), _f32)
    )(h, x, wres, g.reshape(1, D), b.reshape(1, D))


def _layer(x, src, dst, wq, wk, wv, wres, g, b, rel_ids=None,
           rel_table=None, wr=None):
    if rel_ids is not None:
        q, k, v, rp_tab = _tc_qkv1(x, wq, wk, wv, rel_table, wr)
        q_rows, k_rows, rp_rows = _gather_rows_multi(
            (q, k, rp_tab), (dst, src, rel_ids))
    else:
        q, k, v = _tc_qkv2(x, wq, wk, wv)
        rp_tab = rp_rows = None
        q_rows, k_rows = _gather_rows_multi((q, k), (dst, src))
    e16 = _tc_scores(q_rows, k_rows, rp_rows)
    ev2 = _scatter_vals(v, e16, src, dst)
    ev2b = _scatter_vals(rp_tab, e16, rel_ids, dst) if rp_tab is not None else None
    d2 = _scatter_d(e16, dst)
    feat0, dinv_w = _tc_feat0(ev2, d2, ev2b)
    h = feat0
    for _ in range(HOP):
        m2 = _scatter_vals(h, e16, src, dst)
        h = _tc_hop(m2, dinv_w, feat0)
    return _tc_post(h, x, wres, g, b)


def kernel(ent_ids, rel_ids, edge_index, batch_node_ids, params):
    p = params
    src = edge_index[0].astype(_i32)
    dst = edge_index[1].astype(_i32)
    rel_ids = rel_ids.astype(_i32)

    npad = NW * C * 4
    eidx = jnp.zeros((npad,), _i32).at[:N].set(ent_ids.astype(_i32))
    x = _gather_rows(p['ent_table'], eidx)[:N]

    x = _layer(x, src, dst, p['Wq1'], p['Wk1'], p['Wv1'], p['Wres1'],
               p['g1'], p['b1'], rel_ids=rel_ids,
               rel_table=p['rel_table'], wr=p['Wr1'])
    x = _layer(x, src, dst, p['Wq2'], p['Wk2'], p['Wv2'], p['Wres2'],
               p['g2'], p['b2'])

    bpad = NW * C
    bidx = jnp.zeros((bpad,), _i32).at[:batch_node_ids.shape[0]].set(
        batch_node_ids.astype(_i32))
    return _gather_rows(x, bidx)[:batch_node_ids.shape[0]]


# speedup vs baseline: 1.2201x; 1.0406x over previous
import functools

import jax
import jax.numpy as jnp
from jax import lax
from jax.experimental import pallas as pl
from jax.experimental.pallas import tpu as pltpu
from jax.experimental.pallas import tpu_sc as plsc

N = 10000
E = 320000
D = 128
H = 8
DH = 16
HOP = 4
ALPHA = 0.15

NC = 2
NS = 16
NW = NC * NS
C = 80
NACC = 10240
ZC = 32

_f32 = jnp.float32
_i32 = jnp.int32


def _mesh():
    return plsc.VectorSubcoreMesh(core_axis_name="c", subcore_axis_name="s")


def _worker_id():
    return lax.axis_index("s") * NC + lax.axis_index("c")


def _lane(j):
    return jnp.zeros((16,), _i32) + j


@functools.lru_cache(maxsize=None)
def _sc_gather_kernel(n_rows, tab_lens):
    nt = len(tab_lens)
    per_w = n_rows // NW
    n = per_w // C

    def body(*refs):
        tabs = refs[:nt]
        idxs = refs[nt:2 * nt]
        outs = refs[2 * nt:3 * nt]
        idx_v = [[refs[3 * nt + 2 * t + b] for b in (0, 1)] for t in range(nt)]
        rows_v = [[refs[5 * nt + 2 * t + b] for b in (0, 1)] for t in range(nt)]
        lsem, gsem, osem = refs[7 * nt:7 * nt + 3]
        wid = _worker_id()

        def issue(i, b):
            base = wid * per_w + i * C
            for t in range(nt):
                pltpu.async_copy(idxs[t].at[pl.ds(base, C)], idx_v[t][b], lsem)

        def drain_lin(b):
            for t in range(nt):
                pltpu.make_async_copy(
                    idxs[t].at[pl.ds(0, C)], idx_v[t][b], lsem).wait()

        def gstart(b):
            for t in range(nt):
                pltpu.async_copy(tabs[t].at[idx_v[t][b]], rows_v[t][b], gsem)

        def gdrain(b):
            for t in range(nt):
                pltpu.make_async_copy(
                    tabs[t].at[pl.ds(0, C)], rows_v[t][b], gsem).wait()

        def ostart(i, b):
            base = wid * per_w + i * C
            for t in range(nt):
                pltpu.async_copy(rows_v[t][b], outs[t].at[pl.ds(base, C)], osem)

        def odrain(b):
            for t in range(nt):
                pltpu.make_async_copy(
                    rows_v[t][b], outs[t].at[pl.ds(0, C)], osem).wait()

        if n == 1:
            issue(0, 0)
            drain_lin(0)
            gstart(0)
            gdrain(0)
            ostart(0, 0)
            odrain(0)
            return

        issue(0, 0)
        drain_lin(0)
        gstart(0)
        issue(1, 1)

        def pair(g, carry):
            for b in (0, 1):
                i = 2 * g + b
                o = 1 - b

                @pl.when(i < n)
                def _():
                    gdrain(b)

                    @pl.when(i + 1 < n)
                    def _():
                        drain_lin(o)

                        @pl.when(i >= 1)
                        def _():
                            odrain(o)
                        gstart(o)
                    ostart(i, b)

                    @pl.when(i + 2 < n)
                    def _():
                        issue(i + 2, b)
            return carry

        lax.fori_loop(0, (n + 1) // 2, pair, 0)
        odrain((n - 2) % 2)
        odrain((n - 1) % 2)

    scratch = []
    for _ in range(nt):
        scratch += [pltpu.VMEM((C,), _i32)] * 2
    for _ in range(nt):
        scratch += [pltpu.VMEM((C, D), _f32)] * 2
    scratch += [pltpu.SemaphoreType.DMA] * 3
    return pl.kernel(
        body,
        out_type=[jax.ShapeDtypeStruct((n_rows, D), _f32)] * nt,
        mesh=_mesh(),
        scratch_types=scratch,
        name=f"sc_gather{nt}_{n_rows}",
    )


def _gather_rows(table, idx):
    return _sc_gather_kernel(idx.shape[0], (table.shape[0],))(table, idx)[0]


def _gather_rows_multi(tables, idxs):
    n_rows = idxs[0].shape[0]
    return _sc_gather_kernel(n_rows, tuple(t.shape[0] for t in tables))(
        *tables, *idxs)


@functools.lru_cache(maxsize=None)
def _sc_qk_kernel(with_rel):
    nt = 3 if with_rel else 2
    per_w = E // NW
    n = per_w // C

    def body(*refs):
        tabs = refs[:nt]
        idxs = refs[nt:2 * nt]
        out = refs[2 * nt]
        idx_v = [[refs[2 * nt + 1 + 2 * t + b] for b in (0, 1)]
                 for t in range(nt)]
        rows_v = [[refs[4 * nt + 1 + 2 * t + b] for b in (0, 1)]
                  for t in range(nt)]
        lsem, gsem, osem = refs[6 * nt + 1:6 * nt + 4]
        wid = _worker_id()

        def issue(i, b):
            base = wid * per_w + i * C
            for t in range(nt):
                pltpu.async_copy(idxs[t].at[pl.ds(base, C)], idx_v[t][b], lsem)

        def drain_lin(b):
            for t in range(nt):
                pltpu.make_async_copy(
                    idxs[t].at[pl.ds(0, C)], idx_v[t][b], lsem).wait()

        def gstart(b):
            for t in range(nt):
                pltpu.async_copy(tabs[t].at[idx_v[t][b]], rows_v[t][b], gsem)

        def gdrain(b):
            for t in range(nt):
                pltpu.make_async_copy(
                    tabs[t].at[pl.ds(0, C)], rows_v[t][b], gsem).wait()

        def ostart(i, b):
            base = wid * per_w + i * C
            pltpu.async_copy(rows_v[0][b], out.at[pl.ds(base, C)], osem)

        def odrain(b):
            pltpu.make_async_copy(rows_v[0][b], out.at[pl.ds(0, C)],
                                  osem).wait()

        issue(0, 0)
        drain_lin(0)
        gstart(0)
        issue(1, 1)

        def pair(g, carry):
            for b in (0, 1):
                i = 2 * g + b
                o = 1 - b

                @pl.when(i < n)
                def _():
                    gdrain(b)

                    @pl.when(i + 1 < n)
                    def _():
                        drain_lin(o)

                        @pl.when(i >= 1)
                        def _():
                            odrain(o)
                        gstart(o)

                    def mrow(r, cc):
                        for j in range(D // 16):
                            sl = pl.ds(16 * j, 16)
                            ks = rows_v[1][b][r, sl]
                            if with_rel:
                                ks = ks + rows_v[2][b][r, sl]
                            rows_v[0][b][r, sl] = rows_v[0][b][r, sl] * ks
                        return cc

                    lax.fori_loop(0, C, mrow, 0)
                    ostart(i, b)

                    @pl.when(i + 2 < n)
                    def _():
                        issue(i + 2, b)
            return carry

        lax.fori_loop(0, (n + 1) // 2, pair, 0)
        odrain((n - 2) % 2)
        odrain((n - 1) % 2)

    scratch = []
    for _ in range(nt):
        scratch += [pltpu.VMEM((C,), _i32)] * 2
    for _ in range(nt):
        scratch += [pltpu.VMEM((C, D), _f32)] * 2
    scratch += [pltpu.SemaphoreType.DMA] * 3
    return pl.kernel(
        body,
        out_type=jax.ShapeDtypeStruct((E, D), _f32),
        mesh=_mesh(),
        scratch_types=scratch,
        name=f"sc_qk{nt}",
    )


def _qk_rows(q, k, rp_tab, dst, src, rel):
    if rp_tab is not None:
        return _sc_qk_kernel(True)(q, k, rp_tab, dst, src, rel)
    return _sc_qk_kernel(False)(q, k, dst, src)


@functools.lru_cache(maxsize=None)
def _sc_scatter_kernel(tab_len):
    per_w = E // NW
    n = per_w // C
    rpt = NACC // NS

    def body(tab, e16_hbm, src_hbm, dst_hbm, outm,
             sidx0, sidx1, didx0, didx1, e16_0, e16_1, rows_0, rows_1,
             stage, acc, lsem, gsem):
        sidx = [sidx0, sidx1]
        didx = [didx0, didx1]
        e16_v = [e16_0, e16_1]
        rows_v = [rows_0, rows_1]

        cid = lax.axis_index("c")
        sid = lax.axis_index("s")
        wid = sid * NC + cid
        row0 = sid * rpt

        def zrow(r, c):
            for j in range(D // 16):
                stage[r, pl.ds(16 * j, 16)] = jnp.zeros((16,), _f32)
            return c

        lax.fori_loop(0, ZC, zrow, 0)
        for z in range(rpt // ZC):
            pltpu.async_copy(stage.at[:, :], acc.at[pl.ds(row0 + z * ZC, ZC)],
                             lsem)
        for z in range(rpt // ZC):
            pltpu.make_async_copy(stage.at[:, :], acc.at[pl.ds(row0, ZC)],
                                  lsem).wait()
        plsc.subcore_barrier()

        def issue(i, b):
            base = wid * per_w + i * C
            pltpu.async_copy(src_hbm.at[pl.ds(base, C)], sidx[b], lsem)
            pltpu.async_copy(dst_hbm.at[pl.ds(base, C)], didx[b], lsem)
            pltpu.async_copy(e16_hbm.at[pl.ds(base, C)], e16_v[b], lsem)

        def drain_lin(b):
            pltpu.make_async_copy(src_hbm.at[pl.ds(0, C)], sidx[b], lsem).wait()
            pltpu.make_async_copy(dst_hbm.at[pl.ds(0, C)], didx[b], lsem).wait()
            pltpu.make_async_copy(e16_hbm.at[pl.ds(0, C)], e16_v[b], lsem).wait()

        def gstart(b):
            pltpu.async_copy(tab.at[sidx[b]], rows_v[b], gsem)

        def gdrain(b):
            pltpu.make_async_copy(tab.at[pl.ds(0, C)], rows_v[b], gsem).wait()

        issue(0, 0)
        drain_lin(0)
        gstart(0)
        issue(1, 1)

        def pair(g, carry):
            for b in (0, 1):
                i = 2 * g + b
                o = 1 - b

                @pl.when(i < n)
                def _():
                    gdrain(b)

                    @pl.when(i + 1 < n)
                    def _():
                        drain_lin(o)
                        gstart(o)

                    def mrow(r, cc):
                        erow = e16_v[b][r, :]
                        for j in range(H):
                            sl = pl.ds(16 * j, 16)
                            w = jnp.broadcast_to(erow[j], (16,))
                            rows_v[b][r, sl] = rows_v[b][r, sl] * w
                        return cc

                    lax.fori_loop(0, C, mrow, 0)
                    pltpu.sync_copy(rows_v[b], acc.at[didx[b]], add=True)

                    @pl.when(i + 2 < n)
                    def _():
                        issue(i + 2, b)
            return carry

        lax.fori_loop(0, (n + 1) // 2, pair, 0)
        plsc.subcore_barrier()

        pltpu.sync_copy(acc.at[pl.ds(row0, rpt)],
                        outm.at[pl.ds(cid * NACC + row0, rpt)])

    scratch = [pltpu.VMEM((C,), _i32)] * 4
    scratch += [pltpu.VMEM((C, 16), _f32)] * 2
    scratch += [pltpu.VMEM((C, D), _f32)] * 2
    scratch.append(pltpu.VMEM((ZC, D), _f32))
    scratch.append(pltpu.VMEM_SHARED((NACC, D), _f32))
    scratch += [pltpu.SemaphoreType.DMA] * 2
    return pl.kernel(
        body,
        out_type=jax.ShapeDtypeStruct((NC * NACC, D), _f32),
        mesh=_mesh(),
        scratch_types=scratch,
        name="sc_scatter",
    )


@functools.lru_cache(maxsize=None)
def _sc_scatter16_kernel():
    per_w = E // NW
    n = per_w // C
    rpt = NACC // NS
    zc16 = 128

    def body(e16_hbm, dst_hbm, out16, didx0, didx1, e16_0, e16_1,
             stage16, acc16, lsem):
        didx = [didx0, didx1]
        e16_v = [e16_0, e16_1]
        cid = lax.axis_index("c")
        sid = lax.axis_index("s")
        wid = sid * NC + cid
        row0 = sid * rpt

        def zrow16(r, c):
            stage16[r, :] = jnp.zeros((16,), _f32)
            return c

        lax.fori_loop(0, zc16, zrow16, 0)
        for z in range(rpt // zc16):
            pltpu.async_copy(stage16.at[:, :],
                             acc16.at[pl.ds(row0 + z * zc16, zc16)], lsem)
        for z in range(rpt // zc16):
            pltpu.make_async_copy(stage16.at[:, :],
                                  acc16.at[pl.ds(row0, zc16)], lsem).wait()
        plsc.subcore_barrier()

        def issue(i, b):
            base = wid * per_w + i * C
            pltpu.async_copy(dst_hbm.at[pl.ds(base, C)], didx[b], lsem)
            pltpu.async_copy(e16_hbm.at[pl.ds(base, C)], e16_v[b], lsem)

        def drain(b):
            pltpu.make_async_copy(dst_hbm.at[pl.ds(0, C)], didx[b], lsem).wait()
            pltpu.make_async_copy(e16_hbm.at[pl.ds(0, C)], e16_v[b], lsem).wait()

        issue(0, 0)
        issue(1, 1)

        def pair(g, carry):
            for b in (0, 1):
                i = 2 * g + b

                @pl.when(i < n)
                def _():
                    drain(b)
                    pltpu.sync_copy(e16_v[b], acc16.at[didx[b]], add=True)

                    @pl.when(i + 2 < n)
                    def _():
                        issue(i + 2, b)
            return carry

        lax.fori_loop(0, (n + 1) // 2, pair, 0)
        plsc.subcore_barrier()
        pltpu.sync_copy(acc16.at[pl.ds(row0, rpt)],
                        out16.at[pl.ds(cid * NACC + row0, rpt)])

    return pl.kernel(
        body,
        out_type=jax.ShapeDtypeStruct((NC * NACC, 16), _f32),
        mesh=_mesh(),
        scratch_types=[
            pltpu.VMEM((C,), _i32),
            pltpu.VMEM((C,), _i32),
            pltpu.VMEM((C, 16), _f32),
            pltpu.VMEM((C, 16), _f32),
            pltpu.VMEM((zc16, 16), _f32),
            pltpu.VMEM_SHARED((NACC, 16), _f32),
            pltpu.SemaphoreType.DMA,
        ],
        name="sc_scatter_e16",
    )


def _scatter_vals(tab, e16, idx, dst):
    return _sc_scatter_kernel(tab.shape[0])(tab, e16, idx, dst)


def _scatter_d(e16, dst):
    return _sc_scatter16_kernel()(e16, dst)


def _sel(dtype=_f32):
    r = lax.broadcasted_iota(_i32, (D, H), 0) // DH
    c = lax.broadcasted_iota(_i32, (D, H), 1)
    return (r == c).astype(dtype)


def _selT():
    r = lax.broadcasted_iota(_i32, (H, D), 0)
    c = lax.broadcasted_iota(_i32, (H, D), 1) // DH
    return (r == c).astype(_f32)


def _sel16():
    r = lax.broadcasted_iota(_i32, (H, 16), 0)
    c = lax.broadcasted_iota(_i32, (H, 16), 1)
    return (r == c).astype(_f32)


def _tc_qkv1(x, wq, wk, wv, rel_table, wr):
    def body(x_r, wq_r, wk_r, wv_r, rt_r, wr_r, q_r, k_r, v_r, rp_r):
        x_ = x_r[...]
        q_r[...] = jnp.dot(x_, wq_r[...], preferred_element_type=_f32)
        k_r[...] = jnp.dot(x_, wk_r[...], preferred_element_type=_f32)
        v_r[...] = jnp.dot(x_, wv_r[...], preferred_element_type=_f32)
        rp_r[...] = jnp.dot(rt_r[...], wr_r[...], preferred_element_type=_f32)

    return pl.pallas_call(
        body,
        out_shape=[jax.ShapeDtypeStruct((N, D), _f32)] * 3
        + [jax.ShapeDtypeStruct(rel_table.shape, _f32)],
    )(x, wq, wk, wv, rel_table, wr)


def _tc_qkv2(x, wq, wk, wv):
    def body(x_r, wq_r, wk_r, wv_r, q_r, k_r, v_r):
        x_ = x_r[...]
        q_r[...] = jnp.dot(x_, wq_r[...], preferred_element_type=_f32)
        k_r[...] = jnp.dot(x_, wk_r[...], preferred_element_type=_f32)
        v_r[...] = jnp.dot(x_, wv_r[...], preferred_element_type=_f32)

    return pl.pallas_call(
        body, out_shape=[jax.ShapeDtypeStruct((N, D), _f32)] * 3
    )(x, wq, wk, wv)


_EB = 2560


def _tc_scores(qk_rows):
    def body(qk_r, e16_r):
        s = jnp.dot(qk_r[...], _sel(), preferred_element_type=_f32) * 0.25
        s = jnp.where(s >= 0, s, 0.2 * s)
        es = jnp.exp(s)
        e16_r[...] = jnp.dot(es, _sel16(), preferred_element_type=_f32)

    grid = E // _EB
    return pl.pallas_call(
        body,
        grid=(grid,),
        in_specs=[pl.BlockSpec((_EB, D), lambda i: (i, 0))],
        out_specs=pl.BlockSpec((_EB, 16), lambda i: (i, 0)),
        out_shape=jax.ShapeDtypeStruct((E, 16), _f32),
    )(qk_rows)


def _tc_feat0(ev2, d2, ev2b=None):
    def body(*refs):
        if ev2b is None:
            ev_r, d_r, f_r, dinv_r = refs
        else:
            ev_r, evb_r, d_r, f_r, dinv_r = refs
        ev = ev_r[...]
        evs = ev[:N] + ev[NACC:NACC + N]
        if ev2b is not None:
            evb = evb_r[...]
            evs = evs + evb[:N] + evb[NACC:NACC + N]
        d16 = d_r[...][:N] + d_r[...][NACC:NACC + N]
        dinv = 1.0 / (d16[:, :H] + 1e-9)
        dinv_w = jnp.dot(dinv, _selT(), preferred_element_type=_f32)
        f_r[...] = evs * dinv_w
        dinv_r[...] = dinv_w

    args = (ev2, d2) if ev2b is None else (ev2, ev2b, d2)
    return pl.pallas_call(
        body, out_shape=[jax.ShapeDtypeStruct((N, D), _f32)] * 2
    )(*args)


def _tc_hop(m2, dinv_w, feat0):
    def body(m_r, di_r, f_r, h_r):
        m = m_r[...]
        ms = m[:N] + m[NACC:NACC + N]
        h_r[...] = (1.0 - ALPHA) * ms * di_r[...] + ALPHA * f_r[...]

    return pl.pallas_call(
        body, out_shape=jax.ShapeDtypeStruct((N, D), _f32)
    )(m2, dinv_w, feat0)


def _tc_post(h, x, wres, g, b):
    def body(h_r, x_r, w_r, g_r, b_r, o_r):
        o = h_r[...] + jnp.dot(x_r[...], w_r[...], preferred_element_type=_f32)
        mu = jnp.mean(o, axis=1, keepdims=True)
        cdev = o - mu
        var = jnp.mean(cdev * cdev, axis=1, keepdims=True)
        o = cdev * lax.rsqrt(var + 1e-5) * g_r[...] + b_r[...]
        o_r[...] = jnp.where(o > 0, o, jnp.exp(jnp.minimum(o, 0.0)) - 1.0)

    return pl.pallas_call(
        body, out_shape=jax.ShapeDtypeStruct((N, D), _f32)
    )(h, x, wres, g.reshape(1, D), b.reshape(1, D))


def _layer(x, src, dst, wq, wk, wv, wres, g, b, rel_ids=None,
           rel_table=None, wr=None):
    if rel_ids is not None:
        q, k, v, rp_tab = _tc_qkv1(x, wq, wk, wv, rel_table, wr)
    else:
        q, k, v = _tc_qkv2(x, wq, wk, wv)
        rp_tab = None
    qk_rows = _qk_rows(q, k, rp_tab, dst, src, rel_ids)
    e16 = _tc_scores(qk_rows)
    ev2 = _scatter_vals(v, e16, src, dst)
    ev2b = _scatter_vals(rp_tab, e16, rel_ids, dst) if rp_tab is not None else None
    d2 = _scatter_d(e16, dst)
    feat0, dinv_w = _tc_feat0(ev2, d2, ev2b)
    h = feat0
    for _ in range(HOP):
        m2 = _scatter_vals(h, e16, src, dst)
        h = _tc_hop(m2, dinv_w, feat0)
    return _tc_post(h, x, wres, g, b)


def kernel(ent_ids, rel_ids, edge_index, batch_node_ids, params):
    p = params
    src = edge_index[0].astype(_i32)
    dst = edge_index[1].astype(_i32)
    rel_ids = rel_ids.astype(_i32)

    npad = NW * C * 4
    eidx = jnp.zeros((npad,), _i32).at[:N].set(ent_ids.astype(_i32))
    x = _gather_rows(p['ent_table'], eidx)[:N]

    x = _layer(x, src, dst, p['Wq1'], p['Wk1'], p['Wv1'], p['Wres1'],
               p['g1'], p['b1'], rel_ids=rel_ids,
               rel_table=p['rel_table'], wr=p['Wr1'])
    x = _layer(x, src, dst, p['Wq2'], p['Wk2'], p['Wv2'], p['Wres2'],
               p['g2'], p['b2'])

    bpad = NW * C
    bidx = jnp.zeros((bpad,), _i32).at[:batch_node_ids.shape[0]].set(
        batch_node_ids.astype(_i32))
    return _gather_rows(x, bidx)[:batch_node_ids.shape[0]]
